# baseline (speedup 1.0000x reference)
import functools

import jax
import jax.numpy as jnp
from jax import lax
from jax.experimental import pallas as pl
from jax.experimental.pallas import tpu as pltpu
from jax.experimental.pallas import tpu_sc as plsc

B, L, D = 16, 4096, 1024
NC, NS = 2, 16
SCW = 512
K = SCW // 128
G = (NC * NS) // K
DCW = 128
NV = DCW // 16
R = 64
PD = 4
MAXC = B * (L // (R * G))

TCW = D
TBLK = 256


def _body(t_hbm, len_hbm, out_hbm, len_v, buf, accv, tmpv, shared,
          sb, sl0, srv, slast, sems):
    cid = lax.axis_index("c")
    sid = lax.axis_index("s")
    g = lax.rem(sid, G)
    dc = cid * (K // NC) + (sid - g) // G
    d0 = pl.multiple_of(dc * DCW, DCW)
    pltpu.sync_copy(len_hbm, len_v.at[pl.ds(0, B)])

    def build_batch(b, j):
        len_b = len_v[pl.ds(b, 16)][0]
        nchunks = (len_b + (R - 1)) // R
        nk = lax.max(0, (nchunks - g + (G - 1)) // G)

        def put(k, j2):
            c = g + k * G
            sb[j2] = b
            sl0[j2] = c * R
            srv[j2] = len_b - c * R
            slast[j2] = (k == nk - 1).astype(jnp.int32)
            return j2 + 1

        return lax.fori_loop(0, nk, put, j)

    t_total = lax.fori_loop(0, B, build_batch, 0)

    def zero_body(i, _):
        accv[i // NV, pl.ds(16 * lax.rem(i, NV), 16)] = jnp.zeros(
            (16,), jnp.float32)
        return 0

    lax.fori_loop(0, B * NV, zero_body, 0, unroll=4)

    def start(j, par):
        l0 = pl.multiple_of(sl0[j], R)
        pltpu.async_copy(
            t_hbm.at[sb[j], pl.ds(l0, R), pl.ds(d0, DCW)],
            buf.at[par], sems.at[par])

    def wait(j, par):
        l0 = pl.multiple_of(sl0[j], R)
        pltpu.make_async_copy(
            t_hbm.at[sb[j], pl.ds(l0, R), pl.ds(d0, DCW)],
            buf.at[par], sems.at[par]).wait()

    for i in range(PD - 1):
        @pl.when(i < t_total)
        def _():
            start(i, i)

    zero = jnp.zeros((16,), jnp.float32)

    def flat_body(j, acc):
        par = lax.rem(j, PD)
        wait(j, par)

        @pl.when(j + (PD - 1) < t_total)
        def _():
            start(j + (PD - 1), lax.rem(j + (PD - 1), PD))

        rows_valid = srv[j]

        def row_body(r, a):
            m = jnp.broadcast_to(
                (r < rows_valid).astype(jnp.float32), (16,))
            return tuple(
                a[x] + buf[par, r, pl.ds(16 * x, 16)] * m
                for x in range(NV))

        acc = lax.fori_loop(0, R, row_body, acc, unroll=4)
        b = sb[j]
        last = slast[j]

        @pl.when(last == 1)
        def _():
            for x in range(NV):
                accv[b, pl.ds(16 * x, 16)] = acc[x]

        keep = jnp.broadcast_to((last == 0).astype(jnp.float32), (16,))
        return tuple(a * keep for a in acc)

    lax.fori_loop(0, t_total, flat_body, (zero,) * NV)

    pltpu.sync_copy(accv, shared.at[sid])
    plsc.subcore_barrier()

    @pl.when(g == 0)
    def _():
        for j in range(1, G):
            pltpu.sync_copy(shared.at[sid + j], tmpv)

            def add_body(i, _):
                o = pl.ds(16 * lax.rem(i, NV), 16)
                r = i // NV
                accv[r, o] = accv[r, o] + tmpv[r, o]
                return 0

            lax.fori_loop(0, B * NV, add_body, 0, unroll=4)

        def div_body(b, _):
            len_b = len_v[pl.ds(b, 16)][0]
            denom = jnp.broadcast_to(len_b.astype(jnp.float32), (16,))
            for x in range(NV):
                accv[b, pl.ds(16 * x, 16)] = accv[b, pl.ds(16 * x, 16)] / denom
            return 0

        lax.fori_loop(0, B, div_body, 0)
        pltpu.sync_copy(accv, out_hbm.at[:, pl.ds(d0, DCW)])


def _sc_pooled(tensor, lengths):
    mesh = plsc.VectorSubcoreMesh(
        core_axis_name="c", subcore_axis_name="s",
        num_cores=NC, num_subcores=NS)
    f = pl.kernel(
        _body,
        out_type=jax.ShapeDtypeStruct((B, SCW), jnp.float32),
        mesh=mesh,
        scratch_types=[
            pltpu.VMEM((B + 16,), jnp.int32),
            pltpu.VMEM((PD, R, DCW), jnp.float32),
            pltpu.VMEM((B, DCW), jnp.float32),
            pltpu.VMEM((B, DCW), jnp.float32),
            pltpu.VMEM_SHARED((NS, B, DCW), jnp.float32),
            pltpu.SMEM((MAXC,), jnp.int32),
            pltpu.SMEM((MAXC,), jnp.int32),
            pltpu.SMEM((MAXC,), jnp.int32),
            pltpu.SMEM((MAXC,), jnp.int32),
            pltpu.SemaphoreType.DMA((PD,)),
        ],
    )
    return f(tensor, lengths)


def _tc_body(len_ref, x_ref, o_ref):
    j = pl.program_id(1)
    len_b = len_ref[pl.program_id(0)]

    @pl.when(j == 0)
    def _():
        o_ref[...] = jnp.zeros_like(o_ref)

    @pl.when(j * TBLK < len_b)
    def _():
        rows = len_b - j * TBLK
        m = (lax.broadcasted_iota(jnp.int32, (TBLK, 1), 0)
             < rows).astype(jnp.float32)
        o_ref[...] += jnp.sum(x_ref[0] * m, axis=0)[None, None, :]

    @pl.when(j == (L // TBLK) - 1)
    def _():
        o_ref[...] = o_ref[...] / len_b.astype(jnp.float32)


def _tc_pooled(tensor, lengths):
    grid_spec = pltpu.PrefetchScalarGridSpec(
        num_scalar_prefetch=1,
        grid=(B, L // TBLK),
        in_specs=[
            pl.BlockSpec(
                (1, TBLK, TCW),
                lambda b, j, lens: (
                    b,
                    jnp.minimum(j, (lens[b] + (TBLK - 1)) // TBLK - 1),
                    0),
            ),
        ],
        out_specs=pl.BlockSpec((1, 1, TCW), lambda b, j, lens: (b, 0, 0)),
    )
    out3 = pl.pallas_call(
        _tc_body,
        grid_spec=grid_spec,
        out_shape=jax.ShapeDtypeStruct((B, 1, TCW), jnp.float32),
        compiler_params=pltpu.CompilerParams(
            dimension_semantics=("arbitrary", "arbitrary")),
    )(lengths, tensor)
    return out3.reshape(B, TCW)


@jax.jit
def _pooled(tensor, lengths):
    return _tc_pooled(tensor, lengths)


def kernel(tensor, lengths):
    return _pooled(tensor, lengths.astype(jnp.int32))


# speedup vs baseline: 1.4189x; 1.4189x over previous
import functools

import jax
import jax.numpy as jnp
from jax import lax
from jax.experimental import pallas as pl
from jax.experimental.pallas import tpu as pltpu
from jax.experimental.pallas import tpu_sc as plsc

B, L, D = 16, 4096, 1024
NC, NS = 2, 16
SCW = 512
K = SCW // 128
G = (NC * NS) // K
DCW = 128
NV = DCW // 16
R = 64
PD = 4
MAXC = B * (L // (R * G))

TCW = D - SCW
TBLK = 1024


def _body(t_hbm, len_hbm, out_hbm, len_v, buf, accv, tmpv, shared,
          sb, sl0, srv, slast, sems):
    cid = lax.axis_index("c")
    sid = lax.axis_index("s")
    g = lax.rem(sid, G)
    dc = cid * (K // NC) + (sid - g) // G
    d0 = pl.multiple_of(dc * DCW, DCW)
    pltpu.sync_copy(len_hbm, len_v.at[pl.ds(0, B)])

    def build_batch(b, j):
        len_b = len_v[pl.ds(b, 16)][0]
        nchunks = (len_b + (R - 1)) // R
        nk = lax.max(0, (nchunks - g + (G - 1)) // G)

        def put(k, j2):
            c = g + k * G
            sb[j2] = b
            sl0[j2] = c * R
            srv[j2] = len_b - c * R
            slast[j2] = (k == nk - 1).astype(jnp.int32)
            return j2 + 1

        return lax.fori_loop(0, nk, put, j)

    t_total = lax.fori_loop(0, B, build_batch, 0)

    def zero_body(i, _):
        accv[i // NV, pl.ds(16 * lax.rem(i, NV), 16)] = jnp.zeros(
            (16,), jnp.float32)
        return 0

    lax.fori_loop(0, B * NV, zero_body, 0, unroll=4)

    def start(j, par):
        l0 = pl.multiple_of(sl0[j], R)
        pltpu.async_copy(
            t_hbm.at[sb[j], pl.ds(l0, R), pl.ds(d0, DCW)],
            buf.at[par], sems.at[par])

    def wait(j, par):
        l0 = pl.multiple_of(sl0[j], R)
        pltpu.make_async_copy(
            t_hbm.at[sb[j], pl.ds(l0, R), pl.ds(d0, DCW)],
            buf.at[par], sems.at[par]).wait()

    for i in range(PD - 1):
        @pl.when(i < t_total)
        def _():
            start(i, i)

    zero = jnp.zeros((16,), jnp.float32)

    def flat_body(j, acc):
        par = lax.rem(j, PD)
        wait(j, par)

        @pl.when(j + (PD - 1) < t_total)
        def _():
            start(j + (PD - 1), lax.rem(j + (PD - 1), PD))

        rows_valid = srv[j]

        def row_body(r, a):
            m = jnp.broadcast_to(
                (r < rows_valid).astype(jnp.float32), (16,))
            return tuple(
                a[x] + buf[par, r, pl.ds(16 * x, 16)] * m
                for x in range(NV))

        acc = lax.fori_loop(0, R, row_body, acc, unroll=4)
        b = sb[j]
        last = slast[j]

        @pl.when(last == 1)
        def _():
            for x in range(NV):
                accv[b, pl.ds(16 * x, 16)] = acc[x]

        keep = jnp.broadcast_to((last == 0).astype(jnp.float32), (16,))
        return tuple(a * keep for a in acc)

    lax.fori_loop(0, t_total, flat_body, (zero,) * NV)

    pltpu.sync_copy(accv, shared.at[sid])
    plsc.subcore_barrier()

    @pl.when(g == 0)
    def _():
        for j in range(1, G):
            pltpu.sync_copy(shared.at[sid + j], tmpv)

            def add_body(i, _):
                o = pl.ds(16 * lax.rem(i, NV), 16)
                r = i // NV
                accv[r, o] = accv[r, o] + tmpv[r, o]
                return 0

            lax.fori_loop(0, B * NV, add_body, 0, unroll=4)

        def div_body(b, _):
            len_b = len_v[pl.ds(b, 16)][0]
            denom = jnp.broadcast_to(len_b.astype(jnp.float32), (16,))
            for x in range(NV):
                accv[b, pl.ds(16 * x, 16)] = accv[b, pl.ds(16 * x, 16)] / denom
            return 0

        lax.fori_loop(0, B, div_body, 0)
        pltpu.sync_copy(accv, out_hbm.at[:, pl.ds(d0, DCW)])


def _sc_pooled(tensor, lengths):
    mesh = plsc.VectorSubcoreMesh(
        core_axis_name="c", subcore_axis_name="s",
        num_cores=NC, num_subcores=NS)
    f = pl.kernel(
        _body,
        out_type=jax.ShapeDtypeStruct((B, SCW), jnp.float32),
        mesh=mesh,
        scratch_types=[
            pltpu.VMEM((B + 16,), jnp.int32),
            pltpu.VMEM((PD, R, DCW), jnp.float32),
            pltpu.VMEM((B, DCW), jnp.float32),
            pltpu.VMEM((B, DCW), jnp.float32),
            pltpu.VMEM_SHARED((NS, B, DCW), jnp.float32),
            pltpu.SMEM((MAXC,), jnp.int32),
            pltpu.SMEM((MAXC,), jnp.int32),
            pltpu.SMEM((MAXC,), jnp.int32),
            pltpu.SMEM((MAXC,), jnp.int32),
            pltpu.SemaphoreType.DMA((PD,)),
        ],
    )
    return f(tensor, lengths)


def _tc_body(len_ref, x_ref, o_ref):
    j = pl.program_id(1)
    len_b = len_ref[pl.program_id(0)]

    @pl.when(j == 0)
    def _():
        o_ref[...] = jnp.zeros_like(o_ref)

    @pl.when(j * TBLK < len_b)
    def _():
        rows = len_b - j * TBLK
        m = (lax.broadcasted_iota(jnp.int32, (TBLK, 1), 0)
             < rows).astype(jnp.float32)
        o_ref[...] += jnp.sum(x_ref[0] * m, axis=0)[None, None, :]

    @pl.when(j == (L // TBLK) - 1)
    def _():
        o_ref[...] = o_ref[...] / len_b.astype(jnp.float32)


def _tc_pooled(tensor, lengths):
    grid_spec = pltpu.PrefetchScalarGridSpec(
        num_scalar_prefetch=1,
        grid=(B, L // TBLK),
        in_specs=[
            pl.BlockSpec(
                (1, TBLK, TCW),
                lambda b, j, lens: (
                    b,
                    jnp.minimum(j, (lens[b] + (TBLK - 1)) // TBLK - 1),
                    SCW // TCW),
            ),
        ],
        out_specs=pl.BlockSpec((1, 1, TCW), lambda b, j, lens: (b, 0, 0)),
    )
    out3 = pl.pallas_call(
        _tc_body,
        grid_spec=grid_spec,
        out_shape=jax.ShapeDtypeStruct((B, 1, TCW), jnp.float32),
        compiler_params=pltpu.CompilerParams(
            dimension_semantics=("arbitrary", "arbitrary")),
    )(lengths, tensor)
    return out3.reshape(B, TCW)


@jax.jit
def _pooled(tensor, lengths):
    sc_out = _sc_pooled(tensor, lengths)
    tc_out = _tc_pooled(tensor, lengths)
    return jnp.concatenate([sc_out, tc_out], axis=1)


def kernel(tensor, lengths):
    return _pooled(tensor, lengths.astype(jnp.int32))
